# baseline (speedup 1.0000x reference)
import functools

import jax
import jax.numpy as jnp
from jax import lax
from jax.experimental import pallas as pl
from jax.experimental.pallas import tpu as pltpu
from jax.experimental.pallas import tpu_sc as plsc

VOCAB_N = 100000
EMB_N = 300
B_N = 4096
L_N = 200

NC = 2
NS = 16
LANES = 16
NW = NC * NS

SCP_PER = 2176
V_SC = NW * SCP_PER
VT = VOCAB_N - V_SC
SCP_GROUPS = SCP_PER // 16
SCP_PAIRS = SCP_GROUPS // 2

CHUNK = 4096
NCHUNK = (VT + CHUNK - 1) // CHUNK
NBUF = 4
_CHUNK_ROWS = [CHUNK] * (NCHUNK - 1) + [VT - CHUNK * (NCHUNK - 1)]
VT_PAD = ((VT + 127) // 128) * 128


def _mm_body(w_ref, t_hbm, o_ref, buf, sems):
    def start(i):
        rows = _CHUNK_ROWS[i]
        pltpu.make_async_copy(
            t_hbm.at[pl.ds(i * CHUNK, rows), :],
            buf.at[i % NBUF, pl.ds(0, rows), :],
            sems.at[i % NBUF]).start()

    for i in range(NBUF):
        start(i)
    for i in range(NCHUNK):
        rows = _CHUNK_ROWS[i]
        pltpu.make_async_copy(
            t_hbm.at[pl.ds(i * CHUNK, rows), :],
            buf.at[i % NBUF, pl.ds(0, rows), :],
            sems.at[i % NBUF]).wait()
        res = lax.dot_general(
            w_ref[...], buf[i % NBUF, pl.ds(0, rows), :],
            dimension_numbers=(((1,), (1,)), ((), ())),
            preferred_element_type=jnp.float32)
        o_ref[pl.ds(i * CHUNK, rows)] = res[0]
        o_ref[pl.ds(VT_PAD + i * CHUNK, rows)] = res[1]
        if i + NBUF < NCHUNK:
            start(i + NBUF)
    o_ref[pl.ds(0, 1)] = jnp.zeros((1,), jnp.float32)
    o_ref[pl.ds(VT_PAD, 1)] = jnp.zeros((1,), jnp.float32)


def _project_table_tc(Wp, table):
    return pl.pallas_call(
        _mm_body,
        in_specs=[
            pl.BlockSpec(memory_space=pltpu.VMEM),
            pl.BlockSpec(memory_space=pl.ANY),
        ],
        out_specs=pl.BlockSpec(memory_space=pltpu.VMEM),
        out_shape=jax.ShapeDtypeStruct((2 * VT_PAD,), jnp.float32),
        scratch_shapes=[
            pltpu.VMEM((NBUF, CHUNK, EMB_N), jnp.float32),
            pltpu.SemaphoreType.DMA((NBUF,)),
        ],
    )(Wp, table)


NREG = EMB_N // LANES - 1


def _make_sc_project():
    mesh = plsc.VectorSubcoreMesh(core_axis_name="c", subcore_axis_name="s")

    @functools.partial(
        pl.kernel,
        mesh=mesh,
        compiler_params=pltpu.CompilerParams(needs_layout_passes=False),
        out_type=jax.ShapeDtypeStruct((2 * V_SC,), jnp.float32),
        scratch_types=[
            pltpu.VMEM((LANES, EMB_N), jnp.float32),
            pltpu.VMEM((LANES, EMB_N), jnp.float32),
            pltpu.VMEM((2, 304), jnp.float32),
            pltpu.VMEM((SCP_PER,), jnp.float32),
            pltpu.VMEM((SCP_PER,), jnp.float32),
            pltpu.SemaphoreType.DMA,
            pltpu.SemaphoreType.DMA,
        ],
    )
    def sc_project(t_hbm, wg_hbm, out_hbm, bufa, bufb, wg_v, c0_v, c1_v,
                   sema, semb):
        wid = lax.axis_index("s") * NC + lax.axis_index("c")
        base = VT + wid * SCP_PER
        pltpu.sync_copy(wg_hbm, wg_v)
        w0 = [wg_v[0, pl.ds(16 * k, 16)] for k in range(NREG + 1)]
        w1 = [wg_v[1, pl.ds(16 * k, 16)] for k in range(NREG + 1)]

        def start(g, buf, sem):
            pltpu.make_async_copy(
                t_hbm.at[pl.ds(base + g * LANES, LANES), :], buf, sem).start()

        def wait(g, buf, sem):
            pltpu.make_async_copy(
                t_hbm.at[pl.ds(base + g * LANES, LANES), :], buf, sem).wait()

        lane = lax.iota(jnp.int32, 16)

        def process(g, buf):
            vec0 = jnp.zeros((LANES,), jnp.float32)
            vec1 = jnp.zeros((LANES,), jnp.float32)
            for r in range(LANES):
                acc0 = buf[r, pl.ds(0, 16)] * w0[0]
                acc1 = buf[r, pl.ds(0, 16)] * w1[0]
                for k in range(1, NREG):
                    v = buf[r, pl.ds(16 * k, 16)]
                    acc0 = acc0 + v * w0[k]
                    acc1 = acc1 + v * w1[k]
                vt = buf[r, pl.ds(EMB_N - 16, 16)]
                acc0 = acc0 + vt * w0[NREG]
                acc1 = acc1 + vt * w1[NREG]
                m = lane == r
                vec0 = jnp.where(m, jnp.sum(acc0), vec0)
                vec1 = jnp.where(m, jnp.sum(acc1), vec1)
            c0_v[pl.ds(g * LANES, LANES)] = vec0
            c1_v[pl.ds(g * LANES, LANES)] = vec1

        start(0, bufa, sema)

        def pair(p, carry):
            g = p * 2
            start(g + 1, bufb, semb)
            wait(g, bufa, sema)
            process(g, bufa)

            @pl.when(p + 1 < SCP_PAIRS)
            def _():
                start(g + 2, bufa, sema)

            wait(g + 1, bufb, semb)
            process(g + 1, bufb)
            return carry

        lax.fori_loop(0, SCP_PAIRS, pair, 0)
        pltpu.sync_copy(c0_v, out_hbm.at[pl.ds(wid * SCP_PER, SCP_PER)])
        pltpu.sync_copy(c1_v, out_hbm.at[pl.ds(V_SC + wid * SCP_PER,
                                               SCP_PER)])

    return sc_project


_sc_project = _make_sc_project()


ROWS_PER_WORKER = B_N // NS
GROUPS_PER_WORKER = ROWS_PER_WORKER // LANES
GROUP_WORDS = LANES * L_N


def _make_sc_gather():
    mesh = plsc.VectorSubcoreMesh(core_axis_name="c", subcore_axis_name="s")

    @functools.partial(
        pl.kernel,
        mesh=mesh,
        compiler_params=pltpu.CompilerParams(needs_layout_passes=False),
        out_type=jax.ShapeDtypeStruct((2, B_N), jnp.float32),
        scratch_types=[
            pltpu.VMEM((VOCAB_N,), jnp.float32),
            pltpu.VMEM((GROUP_WORDS,), jnp.int32),
            pltpu.VMEM((ROWS_PER_WORKER,), jnp.float32),
            pltpu.VMEM((LANES,), jnp.float32),
            pltpu.SemaphoreType.DMA,
            pltpu.SemaphoreType.DMA,
        ],
    )
    def sc_gather(tc_hbm, sc_hbm, x_hbm, bb_hbm, out_hbm,
                  col_v, idx_v, out_v, b_v, sem_tc, sem_sc):
        cls = lax.axis_index("c")
        w2 = lax.axis_index("s")
        tc_cp = pltpu.make_async_copy(
            tc_hbm.at[pl.ds(cls * VT_PAD, VT)], col_v.at[pl.ds(0, VT)],
            sem_tc)
        tc_cp.start()
        sc_cp = pltpu.make_async_copy(
            sc_hbm.at[pl.ds(cls * V_SC, V_SC)], col_v.at[pl.ds(VT, V_SC)],
            sem_sc)
        sc_cp.start()
        pltpu.sync_copy(bb_hbm.at[cls], b_v)
        tc_cp.wait()
        sc_cp.wait()
        bvec = b_v[...]
        rowoff = lax.iota(jnp.int32, 16) * L_N

        def grp(g, carry):
            gbase = (w2 * GROUPS_PER_WORKER + g) * GROUP_WORDS
            pltpu.sync_copy(x_hbm.at[pl.ds(gbase, GROUP_WORDS)], idx_v)
            acc = jnp.zeros((LANES,), jnp.float32)
            for j in range(L_N):
                idxs = plsc.load_gather(idx_v, [rowoff + j])
                acc = acc + plsc.load_gather(col_v, [idxs])
            z = acc * jnp.float32(1.0 / L_N) + bvec
            out_v[pl.ds(g * LANES, LANES)] = (
                jnp.float32(1.0) / (jnp.float32(1.0) + jnp.exp(-z)))
            return carry

        lax.fori_loop(0, GROUPS_PER_WORKER, grp, 0)
        pltpu.sync_copy(out_v, out_hbm.at[cls, pl.ds(w2 * ROWS_PER_WORKER,
                                                     ROWS_PER_WORKER)])

    return sc_gather


_sc_gather = _make_sc_gather()


def kernel(x, table, W, b):
    xi = x.astype(jnp.int32).reshape(-1)
    tf = table.astype(jnp.float32)
    Wf = W.astype(jnp.float32)
    Wp = jnp.pad(Wf, ((0, 8 - Wf.shape[0]), (0, 0)))
    Wg = jnp.concatenate(
        [Wf[:, :288], jnp.zeros((2, 4), jnp.float32), Wf[:, 288:]], axis=1)
    small_tc = _project_table_tc(Wp, tf)
    small_sc = _sc_project(tf, Wg)
    bb = jnp.broadcast_to(b.astype(jnp.float32)[:, None], (2, LANES))
    out2 = _sc_gather(small_tc, small_sc, xi, bb)
    return out2.T
